# Initial kernel scaffold; baseline (speedup 1.0000x reference)
import jax
import jax.numpy as jnp
from jax.experimental import pallas as pl


def kernel(particles, weights_log, u0, W1, w2):
    raise NotImplementedError("write your pallas kernel here")


# speedup vs baseline: 2.0873x; 2.0873x over previous
import jax
import jax.numpy as jnp
from jax.experimental import pallas as pl
from jax.experimental.pallas import tpu as pltpu

_B, _N, _D, _H = 128, 8192, 64, 256
_ALPHA = 0.5
_RB = 8
_TN = 1024
_NT = _N // _TN


def _cumsum_last(x):
  n = x.shape[-1]
  d = 1
  while d < n:
    x = x + jnp.concatenate([jnp.zeros_like(x[..., :d]), x[..., :-d]], axis=-1)
    d *= 2
  return x


def _stage1_body(wl_ref, u0_ref, a_ref):
  wl = wl_ref[...]
  u0 = u0_ref[...]
  m = jnp.max(wl, axis=-1, keepdims=True)
  e = jnp.exp(wl - m)
  s = jnp.sum(e, axis=-1, keepdims=True)
  probs = e / s
  soft = _ALPHA * probs + (1.0 - _ALPHA) / _N
  cum = _cumsum_last(soft)
  cum = cum / cum[..., -1:]
  t = cum * _N
  k = jnp.floor(t)
  g = k + jnp.where(k + u0 <= t, 1.0, 0.0)
  g = jnp.clip(g, 0.0, float(_N))
  gprev = jnp.concatenate([jnp.zeros_like(g[..., :1]), g[..., :-1]], axis=-1)
  count = g - gprev
  w = probs / (soft + 1e-8) + 1e-8
  a_ref[...] = count * w


def _stage2_body(p_ref, a_ref, w1_ref, w2_ref, o_ref, acc_ref, z_ref):
  j = pl.program_id(1)

  @pl.when(j == 0)
  def _():
    acc_ref[...] = jnp.zeros_like(acc_ref)
    z_ref[0, 0] = 0.0

  p = p_ref[0]
  z = jnp.dot(p, w1_ref[...], preferred_element_type=jnp.float32)
  h = jnp.tanh(z)
  lik = jnp.sum(h * w2_ref[...], axis=-1)
  t = a_ref[0, 0] * jnp.exp(lik)
  acc_ref[...] += jnp.sum(t[:, None] * p, axis=0, keepdims=True)
  z_ref[0, 0] += jnp.sum(t)

  @pl.when(j == _NT - 1)
  def _():
    o_ref[...] = (acc_ref[...] / z_ref[0, 0])[None]


def kernel(particles, weights_log, u0, W1, w2):
  a = pl.pallas_call(
      _stage1_body,
      grid=(_B // _RB,),
      in_specs=[
          pl.BlockSpec((_RB, _N), lambda i: (i, 0)),
          pl.BlockSpec((_RB, 1), lambda i: (i, 0)),
      ],
      out_specs=pl.BlockSpec((_RB, _N), lambda i: (i, 0)),
      out_shape=jax.ShapeDtypeStruct((_B, _N), jnp.float32),
  )(weights_log, u0)

  a3 = a.reshape(_B, 1, _N)
  w2r = w2.reshape(1, _H)

  est3 = pl.pallas_call(
      _stage2_body,
      grid=(_B, _NT),
      in_specs=[
          pl.BlockSpec((1, _TN, _D), lambda b, j: (b, j, 0)),
          pl.BlockSpec((1, 1, _TN), lambda b, j: (b, 0, j)),
          pl.BlockSpec((_D, _H), lambda b, j: (0, 0)),
          pl.BlockSpec((1, _H), lambda b, j: (0, 0)),
      ],
      out_specs=pl.BlockSpec((1, 1, _D), lambda b, j: (b, 0, 0)),
      out_shape=jax.ShapeDtypeStruct((_B, 1, _D), jnp.float32),
      scratch_shapes=[
          pltpu.VMEM((1, _D), jnp.float32),
          pltpu.SMEM((1, 1), jnp.float32),
      ],
      compiler_params=pltpu.CompilerParams(
          dimension_semantics=("parallel", "arbitrary")),
  )(particles, a3, W1, w2r)

  return est3.reshape(_B, _D)


# speedup vs baseline: 2.5305x; 1.2124x over previous
import jax
import jax.numpy as jnp
from jax.experimental import pallas as pl
from jax.experimental.pallas import tpu as pltpu

_B, _N, _D, _H = 128, 8192, 64, 256
_ALPHA = 0.5
_RB = 8
_TN = 1024
_NT = _N // _TN


def _cumsum_last(x):
  n = x.shape[-1]
  d = 1
  while d < n:
    x = x + jnp.concatenate([jnp.zeros_like(x[..., :d]), x[..., :-d]], axis=-1)
    d *= 2
  return x


def _stage1_body(wl_ref, u0_ref, a_ref):
  wl = wl_ref[...]
  u0 = u0_ref[...]
  m = jnp.max(wl, axis=-1, keepdims=True)
  e = jnp.exp(wl - m)
  s = jnp.sum(e, axis=-1, keepdims=True)
  probs = e / s
  soft = _ALPHA * probs + (1.0 - _ALPHA) / _N
  cum = _cumsum_last(soft)
  cum = cum / cum[..., -1:]
  t = cum * _N
  k = jnp.floor(t)
  g = k + jnp.where(k + u0 <= t, 1.0, 0.0)
  g = jnp.clip(g, 0.0, float(_N))
  gprev = jnp.concatenate([jnp.zeros_like(g[..., :1]), g[..., :-1]], axis=-1)
  count = g - gprev
  w = probs / (soft + 1e-8) + 1e-8
  a_ref[...] = count * w


def _stage2_body(p_ref, a_ref, w1t_ref, w2_ref, o_ref, acc_ref, z_ref):
  j = pl.program_id(1)

  @pl.when(j == 0)
  def _():
    acc_ref[...] = jnp.zeros_like(acc_ref)
    z_ref[0, 0] = 0.0

  p = p_ref[0]
  z = jax.lax.dot_general(w1t_ref[...], p, (((1,), (1,)), ((), ())),
                          preferred_element_type=jnp.float32)
  h = jnp.tanh(z)
  lik = jnp.dot(w2_ref[...], h, preferred_element_type=jnp.float32)
  t = a_ref[0] * jnp.exp(lik)
  acc_ref[...] += jnp.dot(t, p, preferred_element_type=jnp.float32)
  z_ref[0, 0] += jnp.sum(t)

  @pl.when(j == _NT - 1)
  def _():
    o_ref[...] = (acc_ref[...] / z_ref[0, 0])[None]


def kernel(particles, weights_log, u0, W1, w2):
  a = pl.pallas_call(
      _stage1_body,
      grid=(_B // _RB,),
      in_specs=[
          pl.BlockSpec((_RB, _N), lambda i: (i, 0)),
          pl.BlockSpec((_RB, 1), lambda i: (i, 0)),
      ],
      out_specs=pl.BlockSpec((_RB, _N), lambda i: (i, 0)),
      out_shape=jax.ShapeDtypeStruct((_B, _N), jnp.float32),
  )(weights_log, u0)

  a3 = a.reshape(_B, 1, _N)
  w1t = W1.T
  w2r = w2.reshape(1, _H)

  est3 = pl.pallas_call(
      _stage2_body,
      grid=(_B, _NT),
      in_specs=[
          pl.BlockSpec((1, _TN, _D), lambda b, j: (b, j, 0)),
          pl.BlockSpec((1, 1, _TN), lambda b, j: (b, 0, j)),
          pl.BlockSpec((_H, _D), lambda b, j: (0, 0)),
          pl.BlockSpec((1, _H), lambda b, j: (0, 0)),
      ],
      out_specs=pl.BlockSpec((1, 1, _D), lambda b, j: (b, 0, 0)),
      out_shape=jax.ShapeDtypeStruct((_B, 1, _D), jnp.float32),
      scratch_shapes=[
          pltpu.VMEM((1, _D), jnp.float32),
          pltpu.SMEM((1, 1), jnp.float32),
      ],
      compiler_params=pltpu.CompilerParams(
          dimension_semantics=("parallel", "arbitrary")),
  )(particles, a3, w1t, w2r)

  return est3.reshape(_B, _D)


# speedup vs baseline: 4.8246x; 1.9065x over previous
import jax
import jax.numpy as jnp
from jax.experimental import pallas as pl
from jax.experimental.pallas import tpu as pltpu

_B, _N, _D, _H = 128, 8192, 64, 256
_ALPHA = 0.5
_RB = 8
_TN = 8192
_NT = _N // _TN
_TC = 512


def _cumsum_last(x):
  n = x.shape[-1]
  d = 1
  while d < n:
    x = x + jnp.concatenate([jnp.zeros_like(x[..., :d]), x[..., :-d]], axis=-1)
    d *= 2
  return x


def _stage1_body(wl_ref, u0_ref, a_ref):
  wl = wl_ref[...]
  u0 = u0_ref[...]
  m = jnp.max(wl, axis=-1, keepdims=True)
  e = jnp.exp(wl - m)
  s = jnp.sum(e, axis=-1, keepdims=True)
  probs = e / s
  soft = _ALPHA * probs + (1.0 - _ALPHA) / _N
  cum = _cumsum_last(soft)
  cum = cum / cum[..., -1:]
  t = cum * _N
  k = jnp.floor(t)
  g = k + jnp.where(k + u0 <= t, 1.0, 0.0)
  g = jnp.clip(g, 0.0, float(_N))
  gprev = jnp.concatenate([jnp.zeros_like(g[..., :1]), g[..., :-1]], axis=-1)
  count = g - gprev
  w = probs / (soft + 1e-8) + 1e-8
  a_ref[...] = count * w


def _stage2_body(p_ref, a_ref, w1t_ref, w2_ref, o_ref, acc_ref, z_ref):
  j = pl.program_id(1)

  @pl.when(j == 0)
  def _():
    acc_ref[...] = jnp.zeros_like(acc_ref)
    z_ref[0, 0] = 0.0

  p = p_ref[0]
  z = jax.lax.dot_general(w1t_ref[...].astype(jnp.bfloat16),
                          p.astype(jnp.bfloat16),
                          (((1,), (1,)), ((), ())),
                          preferred_element_type=jnp.float32)
  h = jnp.tanh(z)
  lik = jnp.dot(w2_ref[...], h, preferred_element_type=jnp.float32)
  t = a_ref[0] * jnp.exp(lik)
  acc_ref[...] += jnp.dot(t, p, preferred_element_type=jnp.float32)
  z_ref[0, 0] += jnp.sum(t)

  @pl.when(j == _NT - 1)
  def _():
    o_ref[...] = (acc_ref[...] / z_ref[0, 0])[None]


def kernel(particles, weights_log, u0, W1, w2):
  a = pl.pallas_call(
      _stage1_body,
      grid=(_B // _RB,),
      in_specs=[
          pl.BlockSpec((_RB, _N), lambda i: (i, 0)),
          pl.BlockSpec((_RB, 1), lambda i: (i, 0)),
      ],
      out_specs=pl.BlockSpec((_RB, _N), lambda i: (i, 0)),
      out_shape=jax.ShapeDtypeStruct((_B, _N), jnp.float32),
  )(weights_log, u0)

  a3 = a.reshape(_B, 1, _N)
  w1t = W1.T
  w2r = w2.reshape(1, _H)

  est3 = pl.pallas_call(
      _stage2_body,
      grid=(_B, _NT),
      in_specs=[
          pl.BlockSpec((1, _TN, _D), lambda b, j: (b, j, 0)),
          pl.BlockSpec((1, 1, _TN), lambda b, j: (b, 0, j)),
          pl.BlockSpec((_H, _D), lambda b, j: (0, 0)),
          pl.BlockSpec((1, _H), lambda b, j: (0, 0)),
      ],
      out_specs=pl.BlockSpec((1, 1, _D), lambda b, j: (b, 0, 0)),
      out_shape=jax.ShapeDtypeStruct((_B, 1, _D), jnp.float32),
      scratch_shapes=[
          pltpu.VMEM((1, _D), jnp.float32),
          pltpu.SMEM((1, 1), jnp.float32),
      ],
      compiler_params=pltpu.CompilerParams(
          dimension_semantics=("parallel", "arbitrary")),
  )(particles, a3, w1t, w2r)

  return est3.reshape(_B, _D)
